# baseline (speedup 1.0000x reference)
import functools

import jax
import jax.numpy as jnp
from jax import lax
from jax.experimental import pallas as pl
from jax.experimental.pallas import tpu as pltpu
from jax.experimental.pallas import tpu_sc as plsc

N = 10000
E = 320000
NS = 16
EPT = E // NS
SL = 640
NPAD = NS * SL
NPAD2 = 2 * NPAD
VPE = EPT // 16
VPS = SL // 16

_F32 = jnp.float32
_I32 = jnp.int32
_NEG = -3.0e38


def _mm_body(x_ref, wl_ref, wr_ref, o1_ref, o2_ref):
    o1_ref[...] = jnp.dot(x_ref[...], wl_ref[...], preferred_element_type=_F32)
    o2_ref[...] = jnp.dot(x_ref[...], wr_ref[...], preferred_element_type=_F32)


def _proj(x, Wl, Wr):
    return pl.pallas_call(
        _mm_body,
        grid=(10,),
        in_specs=[
            pl.BlockSpec((1000, 128), lambda i: (i, 0)),
            pl.BlockSpec((128, 1), lambda i: (0, 0)),
            pl.BlockSpec((128, 1), lambda i: (0, 0)),
        ],
        out_specs=[
            pl.BlockSpec((1000, 1), lambda i: (i, 0)),
            pl.BlockSpec((1000, 1), lambda i: (i, 0)),
        ],
        out_shape=[
            jax.ShapeDtypeStruct((N, 1), _F32),
            jax.ShapeDtypeStruct((N, 1), _F32),
        ],
    )(x, Wl, Wr)


def _rsqrt_vec(a):
    i = plsc.bitcast(a, _I32)
    i = 0x5F3759DF - (i >> 1)
    y = plsc.bitcast(i, _F32)
    for _ in range(4):
        y = y * (1.5 - 0.5 * a * y * y)
    return y


def _sc_body(xl_hbm, xr_hbm, src_hbm, dst_hbm, par_hbm, out_hbm, alpha_hbm,
             tab_t, fin_t, src_t, dst_t, e_t, s_t, acc_t, rs_buf, ra_buf,
             red_row, red_t, par_t, sh_stage, sh_s, sh_nodes, sh_red):
    t = lax.axis_index("s")
    lanes = lax.broadcasted_iota(_I32, (16,), 0)
    zeros16 = jnp.zeros((16,), _F32)

    pltpu.sync_copy(par_hbm, par_t)
    pltpu.sync_copy(src_hbm.at[pl.ds(t * EPT, EPT)], src_t)
    pltpu.sync_copy(dst_hbm.at[pl.ds(t * EPT, EPT)], dst_t)
    pltpu.sync_copy(xl_hbm, tab_t.at[pl.ds(0, N)])
    pltpu.sync_copy(xr_hbm, tab_t.at[pl.ds(NPAD, N)])

    pv = par_t[...]

    def _take(v, idx):
        return v.at[idx].get(mode="promise_in_bounds")

    def _lane(k):
        return _take(pv, jnp.full((16,), k, _I32))

    def _allmax(v):
        for sh in (1, 2, 4, 8):
            v = jnp.maximum(v, _take(v, lanes ^ sh))
        return v

    def _allsum(v):
        for sh in (1, 2, 4, 8):
            v = v + _take(v, lanes ^ sh)
        return v

    att1 = _lane(0)
    b1 = _lane(1)
    g1 = _lane(2)
    be1 = _lane(3)
    wl2 = _lane(4)
    wr2 = _lane(5)
    att2 = _lane(6)
    b2 = _lane(7)
    g2 = _lane(8)
    be2 = _lane(9)

    def _zero_tables(_):
        @plsc.parallel_loop(0, NPAD // 16, unroll=8)
        def zb(i):
            s_t[pl.ds(i * 16, 16)] = zeros16
            acc_t[pl.ds(i * 16, 16)] = zeros16

    _zero_tables(None)

    @plsc.parallel_loop(0, VPE, unroll=25, carry=jnp.full((16,), _NEG, _F32))
    def mx(i, m):
        sl = pl.ds(i * 16, 16)
        si = src_t[sl]
        di = dst_t[sl]
        a = plsc.load_gather(tab_t, [si])
        b = plsc.load_gather(tab_t, [di + NPAD])
        z = a + b
        e = att1 * jnp.maximum(z, 0.2 * z)
        e_t[sl] = e
        return jnp.maximum(m, e)

    def _global_reduce(vec, rnd):
        red_row[...] = vec
        pltpu.sync_copy(red_row, sh_red.at[pl.ds(rnd * 256 + t * 16, 16)])
        plsc.subcore_barrier()
        pltpu.sync_copy(sh_red.at[pl.ds(rnd * 256, 256)], red_t)

    _global_reduce(_allmax(mx), 0)

    def rmax(k, m):
        return jnp.maximum(m, red_t[pl.ds(k * 16, 16)])

    M1 = lax.fori_loop(0, NS, rmax, jnp.full((16,), _NEG, _F32))

    @plsc.parallel_loop(0, VPE, unroll=25)
    def _p2(i):
        sl = pl.ds(i * 16, 16)
        ex = jnp.exp(e_t[sl] - M1)
        e_t[sl] = ex
        si = src_t[sl]
        di = dst_t[sl]
        a = plsc.load_gather(tab_t, [si])
        plsc.addupdate_scatter(s_t, [di], ex)
        plsc.addupdate_scatter(acc_t, [di], ex * a)

    def _combine_and_norm(att_b, gamma, beta, layer):
        pltpu.sync_copy(s_t, sh_stage.at[t])
        plsc.subcore_barrier()

        @plsc.parallel_loop(0, VPS, unroll=8)
        def _zs(i):
            s_t[pl.ds(i * 16, 16)] = zeros16

        def csum_s(k, _c):
            pltpu.sync_copy(sh_stage.at[k, pl.ds(t * SL, SL)], rs_buf)

            @plsc.parallel_loop(0, VPS, unroll=8)
            def _addv(v):
                sl = pl.ds(v * 16, 16)
                s_t[sl] = s_t[sl] + rs_buf[sl]
            return _c
        lax.fori_loop(0, NS, csum_s, 0)
        plsc.subcore_barrier()

        pltpu.sync_copy(acc_t, sh_stage.at[t])
        plsc.subcore_barrier()

        @plsc.parallel_loop(0, VPS, unroll=8)
        def _za(i):
            acc_t[pl.ds(i * 16, 16)] = zeros16

        def csum_a(k, _c):
            pltpu.sync_copy(sh_stage.at[k, pl.ds(t * SL, SL)], ra_buf)

            @plsc.parallel_loop(0, VPS, unroll=8)
            def _addv(v):
                sl = pl.ds(v * 16, 16)
                acc_t[sl] = acc_t[sl] + ra_buf[sl]
            return _c
        lax.fori_loop(0, NS, csum_a, 0)

        base = t * SL

        @plsc.parallel_loop(0, VPS, unroll=8, carry=(zeros16, zeros16))
        def smsq(v, carry):
            sm, sq = carry
            sl = pl.ds(v * 16, 16)
            hp = acc_t[sl] / (s_t[sl] + 1e-16) + att_b
            gi = base + v * 16 + lanes
            hp = jnp.where(gi < N, hp, 0.0)
            acc_t[sl] = hp
            return sm + hp, sq + hp * hp
        sm, sq = smsq
        svec = jnp.where(lanes == 0, _allsum(sm),
                         jnp.where(lanes == 1, _allsum(sq), 0.0))
        _global_reduce(svec, 1 + 2 * layer)

        def rsum(k, acc):
            return acc + red_t[pl.ds(k * 16, 16)]
        tot = lax.fori_loop(0, NS, rsum, zeros16)
        mu = _take(tot, jnp.zeros((16,), _I32)) / N
        var = _take(tot, jnp.ones((16,), _I32)) / N - mu * mu
        rinv = _rsqrt_vec(var + 1e-5)

        @plsc.parallel_loop(0, VPS, unroll=8)
        def _hmath(v):
            sl = pl.ds(v * 16, 16)
            h = gamma * (acc_t[sl] - mu) * rinv + beta
            acc_t[sl] = jnp.maximum(h, 0.0)

    _combine_and_norm(b1, g1, be1, layer=0)

    pltpu.sync_copy(acc_t.at[pl.ds(0, SL)], sh_nodes.at[pl.ds(t * SL, SL)])
    pltpu.sync_copy(s_t.at[pl.ds(0, SL)], sh_s.at[pl.ds(t * SL, SL)])
    plsc.subcore_barrier()
    pltpu.sync_copy(sh_nodes, tab_t.at[pl.ds(0, NPAD)])
    pltpu.sync_copy(sh_s, fin_t)

    @plsc.parallel_loop(0, VPE, unroll=25)
    def _pa(i):
        sl = pl.ds(i * 16, 16)
        sv = plsc.load_gather(fin_t, [dst_t[sl]])
        e_t[sl] = e_t[sl] / (sv + 1e-16)

    pltpu.sync_copy(e_t, alpha_hbm.at[pl.ds(t * EPT, EPT)])

    _zero_tables(None)

    @plsc.parallel_loop(0, VPE, unroll=25, carry=jnp.full((16,), _NEG, _F32))
    def mx2(i, m):
        sl = pl.ds(i * 16, 16)
        si = src_t[sl]
        di = dst_t[sl]
        hs = plsc.load_gather(tab_t, [si])
        hd = plsc.load_gather(tab_t, [di])
        z = wl2 * hs + wr2 * hd
        e = att2 * jnp.maximum(z, 0.2 * z)
        e_t[sl] = e
        return jnp.maximum(m, e)

    _global_reduce(_allmax(mx2), 2)
    M2 = lax.fori_loop(0, NS, rmax, jnp.full((16,), _NEG, _F32))

    @plsc.parallel_loop(0, VPE, unroll=25)
    def _q2(i):
        sl = pl.ds(i * 16, 16)
        ex = jnp.exp(e_t[sl] - M2)
        si = src_t[sl]
        di = dst_t[sl]
        a = plsc.load_gather(tab_t, [si]) * wl2
        plsc.addupdate_scatter(s_t, [di], ex)
        plsc.addupdate_scatter(acc_t, [di], ex * a)

    _combine_and_norm(b2, g2, be2, layer=1)

    pltpu.sync_copy(acc_t.at[pl.ds(0, SL)], out_hbm.at[pl.ds(t * SL, SL)])


@jax.jit
def _sc_gnn(xl_flat, xr_flat, src, dst, par):
    mesh = plsc.VectorSubcoreMesh(core_axis_name="c", subcore_axis_name="s",
                                  num_cores=1)
    f = functools.partial(
        pl.kernel,
        out_type=[
            jax.ShapeDtypeStruct((NPAD,), _F32),
            jax.ShapeDtypeStruct((E,), _F32),
        ],
        mesh=mesh,
        compiler_params=pltpu.CompilerParams(needs_layout_passes=False),
        scratch_types=[
            pltpu.VMEM((NPAD2,), _F32),
            pltpu.VMEM((NPAD,), _F32),
            pltpu.VMEM((EPT,), _I32),
            pltpu.VMEM((EPT,), _I32),
            pltpu.VMEM((EPT,), _F32),
            pltpu.VMEM((NPAD,), _F32),
            pltpu.VMEM((NPAD,), _F32),
            pltpu.VMEM((SL,), _F32),
            pltpu.VMEM((SL,), _F32),
            pltpu.VMEM((16,), _F32),
            pltpu.VMEM((256,), _F32),
            pltpu.VMEM((16,), _F32),
            pltpu.VMEM_SHARED((NS, NPAD), _F32),
            pltpu.VMEM_SHARED((NPAD,), _F32),
            pltpu.VMEM_SHARED((NPAD,), _F32),
            pltpu.VMEM_SHARED((4 * 256,), _F32),
        ],
    )(_sc_body)
    return f(xl_flat, xr_flat, src, dst, par)


def kernel(x, edge_index, Wl1, Wr1, att1, b1, g1, be1, Wl2, Wr2, att2, b2, g2, be2):
    src = edge_index[0].astype(_I32)
    dst = edge_index[1].astype(_I32)
    xl, xr = _proj(x, Wl1, Wr1)
    par = jnp.concatenate([
        att1, b1, g1, be1,
        jnp.reshape(Wl2, (1,)), jnp.reshape(Wr2, (1,)),
        att2, b2, g2, be2,
        jnp.zeros((6,), _F32),
    ])
    h2_pad, a1 = _sc_gnn(jnp.reshape(xl, (-1,)), jnp.reshape(xr, (-1,)),
                         src, dst, par)
    return (jnp.reshape(h2_pad[:N], (1, N)), a1)


# speedup vs baseline: 1.0912x; 1.0912x over previous
import functools

import jax
import jax.numpy as jnp
from jax import lax
from jax.experimental import pallas as pl
from jax.experimental.pallas import tpu as pltpu
from jax.experimental.pallas import tpu_sc as plsc

N = 10000
E = 320000
NS = 16
EPT = E // NS
SL = 640
NPAD = NS * SL
NPAD2 = 2 * NPAD
VPE = EPT // 16
VPS = SL // 16

_F32 = jnp.float32
_I32 = jnp.int32
_NEG = -3.0e38


def _mm_body(x_ref, wl_ref, wr_ref, o1_ref, o2_ref):
    o1_ref[...] = jnp.dot(x_ref[...], wl_ref[...], preferred_element_type=_F32)
    o2_ref[...] = jnp.dot(x_ref[...], wr_ref[...], preferred_element_type=_F32)


def _proj(x, Wl, Wr):
    return pl.pallas_call(
        _mm_body,
        grid=(10,),
        in_specs=[
            pl.BlockSpec((1000, 128), lambda i: (i, 0)),
            pl.BlockSpec((128, 1), lambda i: (0, 0)),
            pl.BlockSpec((128, 1), lambda i: (0, 0)),
        ],
        out_specs=[
            pl.BlockSpec((1000, 1), lambda i: (i, 0)),
            pl.BlockSpec((1000, 1), lambda i: (i, 0)),
        ],
        out_shape=[
            jax.ShapeDtypeStruct((N, 1), _F32),
            jax.ShapeDtypeStruct((N, 1), _F32),
        ],
    )(x, Wl, Wr)


def _rsqrt_vec(a):
    i = plsc.bitcast(a, _I32)
    i = 0x5F3759DF - (i >> 1)
    y = plsc.bitcast(i, _F32)
    for _ in range(4):
        y = y * (1.5 - 0.5 * a * y * y)
    return y


def _sc_body(xl_hbm, xr_hbm, src_hbm, dst_hbm, par_hbm, out_hbm, alpha_hbm,
             tab_t, fin_t, src_t, dst_t, e_t, s_t, acc_t, rs_buf, ra_buf,
             red_row, red_t, par_t, sh_stage, sh_s, sh_nodes, sh_red):
    t = lax.axis_index("s")
    lanes = lax.broadcasted_iota(_I32, (16,), 0)
    zeros16 = jnp.zeros((16,), _F32)

    pltpu.sync_copy(par_hbm, par_t)
    pltpu.sync_copy(src_hbm.at[pl.ds(t * EPT, EPT)], src_t)
    pltpu.sync_copy(dst_hbm.at[pl.ds(t * EPT, EPT)], dst_t)
    pltpu.sync_copy(xl_hbm, tab_t.at[pl.ds(0, N)])
    pltpu.sync_copy(xr_hbm, tab_t.at[pl.ds(NPAD, N)])

    pv = par_t[...]

    def _take(v, idx):
        return v.at[idx].get(mode="promise_in_bounds")

    def _lane(k):
        return _take(pv, jnp.full((16,), k, _I32))

    def _allmax(v):
        for sh in (1, 2, 4, 8):
            v = jnp.maximum(v, _take(v, lanes ^ sh))
        return v

    def _allsum(v):
        for sh in (1, 2, 4, 8):
            v = v + _take(v, lanes ^ sh)
        return v

    att1 = _lane(0)
    b1 = _lane(1)
    g1 = _lane(2)
    be1 = _lane(3)
    wl2 = _lane(4)
    wr2 = _lane(5)
    att2 = _lane(6)
    b2 = _lane(7)
    g2 = _lane(8)
    be2 = _lane(9)

    def _zero_tables(_):
        @plsc.parallel_loop(0, NPAD // 16, unroll=8)
        def zb(i):
            s_t[pl.ds(i * 16, 16)] = zeros16
            acc_t[pl.ds(i * 16, 16)] = zeros16

    _zero_tables(None)

    def _lk(v):
        return jnp.maximum(v, 0.2 * v)

    def _global_reduce(vec, rnd):
        red_row[...] = vec
        pltpu.sync_copy(red_row, sh_red.at[pl.ds(rnd * 256 + t * 16, 16)])
        plsc.subcore_barrier()
        pltpu.sync_copy(sh_red.at[pl.ds(rnd * 256, 256)], red_t)

    def rmax(k, m):
        return jnp.maximum(m, red_t[pl.ds(k * 16, 16)])

    nb = t * SL
    neg4 = (jnp.full((16,), _NEG, _F32),) * 4

    @plsc.parallel_loop(0, VPS, unroll=8, carry=neg4)
    def xstats(v, carry):
        pxl, nxl, pxr, nxr = carry
        gi = nb + v * 16 + lanes
        ok = gi < N
        xl = tab_t[pl.ds(nb + v * 16, 16)]
        xr = tab_t[pl.ds(NPAD + nb + v * 16, 16)]
        pxl = jnp.maximum(pxl, jnp.where(ok, xl, _NEG))
        nxl = jnp.maximum(nxl, jnp.where(ok, -xl, _NEG))
        pxr = jnp.maximum(pxr, jnp.where(ok, xr, _NEG))
        nxr = jnp.maximum(nxr, jnp.where(ok, -xr, _NEG))
        return pxl, nxl, pxr, nxr

    pxl, nxl, pxr, nxr = xstats
    svec0 = jnp.where(lanes == 0, _allmax(pxl),
                      jnp.where(lanes == 1, _allmax(nxl),
                                jnp.where(lanes == 2, _allmax(pxr),
                                          jnp.where(lanes == 3, _allmax(nxr),
                                                    _NEG))))
    _global_reduce(svec0, 0)
    mm = lax.fori_loop(0, NS, rmax, jnp.full((16,), _NEG, _F32))
    zmax = _take(mm, jnp.zeros((16,), _I32)) + _take(mm, jnp.full((16,), 2, _I32))
    zmin = -(_take(mm, jnp.ones((16,), _I32)) + _take(mm, jnp.full((16,), 3, _I32)))
    M1 = jnp.maximum(att1 * _lk(zmax), att1 * _lk(zmin))

    @plsc.parallel_loop(0, VPE, unroll=10)
    def _p2(i):
        sl = pl.ds(i * 16, 16)
        si = src_t[sl]
        di = dst_t[sl]
        a = plsc.load_gather(tab_t, [si])
        b = plsc.load_gather(tab_t, [di + NPAD])
        ex = jnp.exp(att1 * _lk(a + b) - M1)
        e_t[sl] = ex
        plsc.addupdate_scatter(s_t, [di], ex)
        plsc.addupdate_scatter(acc_t, [di], ex * a)

    def _combine_and_norm(att_b, gamma, beta, layer):
        pltpu.sync_copy(s_t, sh_stage.at[t])
        plsc.subcore_barrier()

        @plsc.parallel_loop(0, VPS, unroll=8)
        def _zs(i):
            s_t[pl.ds(i * 16, 16)] = zeros16

        def csum_s(k, _c):
            pltpu.sync_copy(sh_stage.at[k, pl.ds(t * SL, SL)], rs_buf)

            @plsc.parallel_loop(0, VPS, unroll=8)
            def _addv(v):
                sl = pl.ds(v * 16, 16)
                s_t[sl] = s_t[sl] + rs_buf[sl]
            return _c
        lax.fori_loop(0, NS, csum_s, 0)
        plsc.subcore_barrier()

        pltpu.sync_copy(acc_t, sh_stage.at[t])
        plsc.subcore_barrier()

        @plsc.parallel_loop(0, VPS, unroll=8)
        def _za(i):
            acc_t[pl.ds(i * 16, 16)] = zeros16

        def csum_a(k, _c):
            pltpu.sync_copy(sh_stage.at[k, pl.ds(t * SL, SL)], ra_buf)

            @plsc.parallel_loop(0, VPS, unroll=8)
            def _addv(v):
                sl = pl.ds(v * 16, 16)
                acc_t[sl] = acc_t[sl] + ra_buf[sl]
            return _c
        lax.fori_loop(0, NS, csum_a, 0)

        base = t * SL

        @plsc.parallel_loop(0, VPS, unroll=8, carry=(zeros16, zeros16))
        def smsq(v, carry):
            sm, sq = carry
            sl = pl.ds(v * 16, 16)
            hp = acc_t[sl] / (s_t[sl] + 1e-16) + att_b
            gi = base + v * 16 + lanes
            hp = jnp.where(gi < N, hp, 0.0)
            acc_t[sl] = hp
            return sm + hp, sq + hp * hp
        sm, sq = smsq
        svec = jnp.where(lanes == 0, _allsum(sm),
                         jnp.where(lanes == 1, _allsum(sq), 0.0))
        _global_reduce(svec, 1 + 2 * layer)

        def rsum(k, acc):
            return acc + red_t[pl.ds(k * 16, 16)]
        tot = lax.fori_loop(0, NS, rsum, zeros16)
        mu = _take(tot, jnp.zeros((16,), _I32)) / N
        var = _take(tot, jnp.ones((16,), _I32)) / N - mu * mu
        rinv = _rsqrt_vec(var + 1e-5)

        @plsc.parallel_loop(0, VPS, unroll=8, carry=zeros16)
        def hmx(v, m):
            sl = pl.ds(v * 16, 16)
            h = jnp.maximum(gamma * (acc_t[sl] - mu) * rinv + beta, 0.0)
            acc_t[sl] = h
            return jnp.maximum(m, h)
        return hmx

    hmx1 = _combine_and_norm(b1, g1, be1, layer=0)
    _global_reduce(_allmax(hmx1), 2)
    maxh = lax.fori_loop(0, NS, rmax, jnp.full((16,), _NEG, _F32))

    pltpu.sync_copy(acc_t.at[pl.ds(0, SL)], sh_nodes.at[pl.ds(t * SL, SL)])
    pltpu.sync_copy(s_t.at[pl.ds(0, SL)], sh_s.at[pl.ds(t * SL, SL)])
    plsc.subcore_barrier()
    pltpu.sync_copy(sh_nodes, tab_t.at[pl.ds(0, NPAD)])
    pltpu.sync_copy(sh_s, fin_t)

    @plsc.parallel_loop(0, VPE, unroll=10)
    def _pa(i):
        sl = pl.ds(i * 16, 16)
        sv = plsc.load_gather(fin_t, [dst_t[sl]])
        e_t[sl] = e_t[sl] / (sv + 1e-16)

    pltpu.sync_copy(e_t, alpha_hbm.at[pl.ds(t * EPT, EPT)])

    _zero_tables(None)

    zmax2 = (jnp.maximum(wl2, 0.0) + jnp.maximum(wr2, 0.0)) * maxh
    zmin2 = (jnp.minimum(wl2, 0.0) + jnp.minimum(wr2, 0.0)) * maxh
    M2 = jnp.maximum(att2 * _lk(zmax2), att2 * _lk(zmin2))

    @plsc.parallel_loop(0, VPE, unroll=10)
    def _q2(i):
        sl = pl.ds(i * 16, 16)
        si = src_t[sl]
        di = dst_t[sl]
        hs = plsc.load_gather(tab_t, [si])
        hd = plsc.load_gather(tab_t, [di])
        ex = jnp.exp(att2 * _lk(wl2 * hs + wr2 * hd) - M2)
        plsc.addupdate_scatter(s_t, [di], ex)
        plsc.addupdate_scatter(acc_t, [di], ex * (wl2 * hs))

    _combine_and_norm(b2, g2, be2, layer=1)

    pltpu.sync_copy(acc_t.at[pl.ds(0, SL)], out_hbm.at[pl.ds(t * SL, SL)])


@jax.jit
def _sc_gnn(xl_flat, xr_flat, src, dst, par):
    mesh = plsc.VectorSubcoreMesh(core_axis_name="c", subcore_axis_name="s",
                                  num_cores=1)
    f = functools.partial(
        pl.kernel,
        out_type=[
            jax.ShapeDtypeStruct((NPAD,), _F32),
            jax.ShapeDtypeStruct((E,), _F32),
        ],
        mesh=mesh,
        compiler_params=pltpu.CompilerParams(needs_layout_passes=False),
        scratch_types=[
            pltpu.VMEM((NPAD2,), _F32),
            pltpu.VMEM((NPAD,), _F32),
            pltpu.VMEM((EPT,), _I32),
            pltpu.VMEM((EPT,), _I32),
            pltpu.VMEM((EPT,), _F32),
            pltpu.VMEM((NPAD,), _F32),
            pltpu.VMEM((NPAD,), _F32),
            pltpu.VMEM((SL,), _F32),
            pltpu.VMEM((SL,), _F32),
            pltpu.VMEM((16,), _F32),
            pltpu.VMEM((256,), _F32),
            pltpu.VMEM((16,), _F32),
            pltpu.VMEM_SHARED((NS, NPAD), _F32),
            pltpu.VMEM_SHARED((NPAD,), _F32),
            pltpu.VMEM_SHARED((NPAD,), _F32),
            pltpu.VMEM_SHARED((4 * 256,), _F32),
        ],
    )(_sc_body)
    return f(xl_flat, xr_flat, src, dst, par)


def kernel(x, edge_index, Wl1, Wr1, att1, b1, g1, be1, Wl2, Wr2, att2, b2, g2, be2):
    src = edge_index[0].astype(_I32)
    dst = edge_index[1].astype(_I32)
    xl, xr = _proj(x, Wl1, Wr1)
    par = jnp.concatenate([
        att1, b1, g1, be1,
        jnp.reshape(Wl2, (1,)), jnp.reshape(Wr2, (1,)),
        att2, b2, g2, be2,
        jnp.zeros((6,), _F32),
    ])
    h2_pad, a1 = _sc_gnn(jnp.reshape(xl, (-1,)), jnp.reshape(xr, (-1,)),
                         src, dst, par)
    return (jnp.reshape(h2_pad[:N], (1, N)), a1)


# speedup vs baseline: 1.1237x; 1.0298x over previous
import functools

import jax
import jax.numpy as jnp
from jax import lax
from jax.experimental import pallas as pl
from jax.experimental.pallas import tpu as pltpu
from jax.experimental.pallas import tpu_sc as plsc

N = 10000
E = 320000
NS = 16
EPT = E // NS
SL = 640
NPAD = NS * SL
NPAD2 = 2 * NPAD
VPE = EPT // 16
VPS = SL // 16

_F32 = jnp.float32
_I32 = jnp.int32
_NEG = -3.0e38


def _mm_body(x_ref, wl_ref, wr_ref, o1_ref, o2_ref):
    o1_ref[...] = jnp.dot(x_ref[...], wl_ref[...], preferred_element_type=_F32)
    o2_ref[...] = jnp.dot(x_ref[...], wr_ref[...], preferred_element_type=_F32)


def _proj(x, Wl, Wr):
    return pl.pallas_call(
        _mm_body,
        grid=(10,),
        in_specs=[
            pl.BlockSpec((1000, 128), lambda i: (i, 0)),
            pl.BlockSpec((128, 1), lambda i: (0, 0)),
            pl.BlockSpec((128, 1), lambda i: (0, 0)),
        ],
        out_specs=[
            pl.BlockSpec((1000, 1), lambda i: (i, 0)),
            pl.BlockSpec((1000, 1), lambda i: (i, 0)),
        ],
        out_shape=[
            jax.ShapeDtypeStruct((N, 1), _F32),
            jax.ShapeDtypeStruct((N, 1), _F32),
        ],
    )(x, Wl, Wr)


def _rsqrt_vec(a):
    i = plsc.bitcast(a, _I32)
    i = 0x5F3759DF - (i >> 1)
    y = plsc.bitcast(i, _F32)
    for _ in range(4):
        y = y * (1.5 - 0.5 * a * y * y)
    return y


def _sc_body(xl_hbm, xr_hbm, src_hbm, dst_hbm, par_hbm, out_hbm, alpha_hbm,
             tab_t, fin_t, src_t, dst_t, e_t, s_t, acc_t, rs_buf, ra_buf,
             red_row, red_t, par_t, sem_e, sem_a, sh_stage, sh_s, sh_nodes,
             sh_red):
    t = lax.axis_index("s")
    lanes = lax.broadcasted_iota(_I32, (16,), 0)
    zeros16 = jnp.zeros((16,), _F32)

    d_src = pltpu.async_copy(src_hbm.at[pl.ds(t * EPT, EPT)], src_t, sem_e)
    d_dst = pltpu.async_copy(dst_hbm.at[pl.ds(t * EPT, EPT)], dst_t, sem_e)
    pltpu.sync_copy(par_hbm, par_t)
    pltpu.sync_copy(xl_hbm, tab_t.at[pl.ds(0, N)])
    pltpu.sync_copy(xr_hbm, tab_t.at[pl.ds(NPAD, N)])

    pv = par_t[...]

    def _take(v, idx):
        return v.at[idx].get(mode="promise_in_bounds")

    def _lane(k):
        return _take(pv, jnp.full((16,), k, _I32))

    def _allmax(v):
        for sh in (1, 2, 4, 8):
            v = jnp.maximum(v, _take(v, lanes ^ sh))
        return v

    def _allsum(v):
        for sh in (1, 2, 4, 8):
            v = v + _take(v, lanes ^ sh)
        return v

    att1 = _lane(0)
    b1 = _lane(1)
    g1 = _lane(2)
    be1 = _lane(3)
    wl2 = _lane(4)
    wr2 = _lane(5)
    att2 = _lane(6)
    b2 = _lane(7)
    g2 = _lane(8)
    be2 = _lane(9)

    def _zero_tables(_):
        @plsc.parallel_loop(0, NPAD // 16, unroll=8)
        def zb(i):
            s_t[pl.ds(i * 16, 16)] = zeros16
            acc_t[pl.ds(i * 16, 16)] = zeros16

    _zero_tables(None)

    def _lk(v):
        return jnp.maximum(v, 0.2 * v)

    def _global_reduce(vec, rnd):
        red_row[...] = vec
        pltpu.sync_copy(red_row, sh_red.at[pl.ds(rnd * 256 + t * 16, 16)])
        plsc.subcore_barrier()
        pltpu.sync_copy(sh_red.at[pl.ds(rnd * 256, 256)], red_t)

    def rmax(k, m):
        return jnp.maximum(m, red_t[pl.ds(k * 16, 16)])

    nb = t * SL
    neg4 = (jnp.full((16,), _NEG, _F32),) * 4

    @plsc.parallel_loop(0, VPS, unroll=8, carry=neg4)
    def xstats(v, carry):
        pxl, nxl, pxr, nxr = carry
        gi = nb + v * 16 + lanes
        ok = gi < N
        xl = tab_t[pl.ds(nb + v * 16, 16)]
        xr = tab_t[pl.ds(NPAD + nb + v * 16, 16)]
        pxl = jnp.maximum(pxl, jnp.where(ok, xl, _NEG))
        nxl = jnp.maximum(nxl, jnp.where(ok, -xl, _NEG))
        pxr = jnp.maximum(pxr, jnp.where(ok, xr, _NEG))
        nxr = jnp.maximum(nxr, jnp.where(ok, -xr, _NEG))
        return pxl, nxl, pxr, nxr

    pxl, nxl, pxr, nxr = xstats
    svec0 = jnp.where(lanes == 0, _allmax(pxl),
                      jnp.where(lanes == 1, _allmax(nxl),
                                jnp.where(lanes == 2, _allmax(pxr),
                                          jnp.where(lanes == 3, _allmax(nxr),
                                                    _NEG))))
    _global_reduce(svec0, 0)
    mm = lax.fori_loop(0, NS, rmax, jnp.full((16,), _NEG, _F32))
    zmax = _take(mm, jnp.zeros((16,), _I32)) + _take(mm, jnp.full((16,), 2, _I32))
    zmin = -(_take(mm, jnp.ones((16,), _I32)) + _take(mm, jnp.full((16,), 3, _I32)))
    M1 = jnp.maximum(att1 * _lk(zmax), att1 * _lk(zmin))
    d_src.wait()
    d_dst.wait()

    @plsc.parallel_loop(0, VPE, unroll=10)
    def _p2(i):
        sl = pl.ds(i * 16, 16)
        si = src_t[sl]
        di = dst_t[sl]
        a = plsc.load_gather(tab_t, [si])
        b = plsc.load_gather(tab_t, [di + NPAD])
        ex = jnp.exp(att1 * _lk(a + b) - M1)
        e_t[sl] = ex
        plsc.addupdate_scatter(s_t, [di], ex)
        plsc.addupdate_scatter(acc_t, [di], ex * a)

    def _combine_and_norm(att_b, gamma, beta, layer):
        pltpu.sync_copy(s_t, sh_stage.at[t])
        plsc.subcore_barrier()

        @plsc.parallel_loop(0, VPS, unroll=8)
        def _zs(i):
            s_t[pl.ds(i * 16, 16)] = zeros16

        def csum_s(k, _c):
            pltpu.sync_copy(sh_stage.at[k, pl.ds(t * SL, SL)], rs_buf)

            @plsc.parallel_loop(0, VPS, unroll=8)
            def _addv(v):
                sl = pl.ds(v * 16, 16)
                s_t[sl] = s_t[sl] + rs_buf[sl]
            return _c
        lax.fori_loop(0, NS, csum_s, 0)
        plsc.subcore_barrier()

        pltpu.sync_copy(acc_t, sh_stage.at[t])
        plsc.subcore_barrier()

        @plsc.parallel_loop(0, VPS, unroll=8)
        def _za(i):
            acc_t[pl.ds(i * 16, 16)] = zeros16

        def csum_a(k, _c):
            pltpu.sync_copy(sh_stage.at[k, pl.ds(t * SL, SL)], ra_buf)

            @plsc.parallel_loop(0, VPS, unroll=8)
            def _addv(v):
                sl = pl.ds(v * 16, 16)
                acc_t[sl] = acc_t[sl] + ra_buf[sl]
            return _c
        lax.fori_loop(0, NS, csum_a, 0)

        base = t * SL

        @plsc.parallel_loop(0, VPS, unroll=8, carry=(zeros16, zeros16))
        def smsq(v, carry):
            sm, sq = carry
            sl = pl.ds(v * 16, 16)
            hp = acc_t[sl] / (s_t[sl] + 1e-16) + att_b
            gi = base + v * 16 + lanes
            hp = jnp.where(gi < N, hp, 0.0)
            acc_t[sl] = hp
            return sm + hp, sq + hp * hp
        sm, sq = smsq
        svec = jnp.where(lanes == 0, _allsum(sm),
                         jnp.where(lanes == 1, _allsum(sq), 0.0))
        _global_reduce(svec, 1 + 2 * layer)

        def rsum(k, acc):
            return acc + red_t[pl.ds(k * 16, 16)]
        tot = lax.fori_loop(0, NS, rsum, zeros16)
        mu = _take(tot, jnp.zeros((16,), _I32)) / N
        var = _take(tot, jnp.ones((16,), _I32)) / N - mu * mu
        rinv = _rsqrt_vec(var + 1e-5)

        @plsc.parallel_loop(0, VPS, unroll=8, carry=zeros16)
        def hmx(v, m):
            sl = pl.ds(v * 16, 16)
            h = jnp.maximum(gamma * (acc_t[sl] - mu) * rinv + beta, 0.0)
            acc_t[sl] = h
            return jnp.maximum(m, h)
        return hmx

    hmx1 = _combine_and_norm(b1, g1, be1, layer=0)
    _global_reduce(_allmax(hmx1), 2)
    maxh = lax.fori_loop(0, NS, rmax, jnp.full((16,), _NEG, _F32))

    pltpu.sync_copy(acc_t.at[pl.ds(0, SL)], sh_nodes.at[pl.ds(t * SL, SL)])
    pltpu.sync_copy(s_t.at[pl.ds(0, SL)], sh_s.at[pl.ds(t * SL, SL)])
    plsc.subcore_barrier()
    pltpu.sync_copy(sh_nodes, tab_t.at[pl.ds(0, NPAD)])
    pltpu.sync_copy(sh_s, fin_t)

    @plsc.parallel_loop(0, VPE, unroll=10)
    def _pa(i):
        sl = pl.ds(i * 16, 16)
        sv = plsc.load_gather(fin_t, [dst_t[sl]])
        e_t[sl] = e_t[sl] / (sv + 1e-16)

    d_alpha = pltpu.async_copy(e_t, alpha_hbm.at[pl.ds(t * EPT, EPT)], sem_a)

    _zero_tables(None)

    zmax2 = (jnp.maximum(wl2, 0.0) + jnp.maximum(wr2, 0.0)) * maxh
    zmin2 = (jnp.minimum(wl2, 0.0) + jnp.minimum(wr2, 0.0)) * maxh
    M2 = jnp.maximum(att2 * _lk(zmax2), att2 * _lk(zmin2))

    @plsc.parallel_loop(0, VPE, unroll=10)
    def _q2(i):
        sl = pl.ds(i * 16, 16)
        si = src_t[sl]
        di = dst_t[sl]
        hs = plsc.load_gather(tab_t, [si])
        hd = plsc.load_gather(tab_t, [di])
        ex = jnp.exp(att2 * _lk(wl2 * hs + wr2 * hd) - M2)
        plsc.addupdate_scatter(s_t, [di], ex)
        plsc.addupdate_scatter(acc_t, [di], ex * (wl2 * hs))

    _combine_and_norm(b2, g2, be2, layer=1)

    @pl.when(t == NS - 1)
    def _tail():
        pltpu.sync_copy(acc_t.at[pl.ds(0, N - (NS - 1) * SL)],
                        out_hbm.at[pl.ds((NS - 1) * SL, N - (NS - 1) * SL)])

    @pl.when(t != NS - 1)
    def _body():
        pltpu.sync_copy(acc_t.at[pl.ds(0, SL)], out_hbm.at[pl.ds(t * SL, SL)])

    d_alpha.wait()


@jax.jit
def _sc_gnn(xl_flat, xr_flat, src, dst, par):
    mesh = plsc.VectorSubcoreMesh(core_axis_name="c", subcore_axis_name="s",
                                  num_cores=1)
    f = functools.partial(
        pl.kernel,
        out_type=[
            jax.ShapeDtypeStruct((N,), _F32),
            jax.ShapeDtypeStruct((E,), _F32),
        ],
        mesh=mesh,
        compiler_params=pltpu.CompilerParams(needs_layout_passes=False),
        scratch_types=[
            pltpu.VMEM((NPAD2,), _F32),
            pltpu.VMEM((NPAD,), _F32),
            pltpu.VMEM((EPT,), _I32),
            pltpu.VMEM((EPT,), _I32),
            pltpu.VMEM((EPT,), _F32),
            pltpu.VMEM((NPAD,), _F32),
            pltpu.VMEM((NPAD,), _F32),
            pltpu.VMEM((SL,), _F32),
            pltpu.VMEM((SL,), _F32),
            pltpu.VMEM((16,), _F32),
            pltpu.VMEM((256,), _F32),
            pltpu.VMEM((16,), _F32),
            pltpu.SemaphoreType.DMA,
            pltpu.SemaphoreType.DMA,
            pltpu.VMEM_SHARED((NS, NPAD), _F32),
            pltpu.VMEM_SHARED((NPAD,), _F32),
            pltpu.VMEM_SHARED((NPAD,), _F32),
            pltpu.VMEM_SHARED((4 * 256,), _F32),
        ],
    )(_sc_body)
    return f(xl_flat, xr_flat, src, dst, par)


def kernel(x, edge_index, Wl1, Wr1, att1, b1, g1, be1, Wl2, Wr2, att2, b2, g2, be2):
    src = edge_index[0].astype(_I32)
    dst = edge_index[1].astype(_I32)
    xl, xr = _proj(x, Wl1, Wr1)
    par = jnp.concatenate([
        att1, b1, g1, be1,
        jnp.reshape(Wl2, (1,)), jnp.reshape(Wr2, (1,)),
        att2, b2, g2, be2,
        jnp.zeros((6,), _F32),
    ])
    h2, a1 = _sc_gnn(jnp.reshape(xl, (-1,)), jnp.reshape(xr, (-1,)),
                     src, dst, par)
    return (jnp.reshape(h2, (1, N)), a1)
